# baseline (speedup 1.0000x reference)
import functools

import jax
import jax.numpy as jnp
from jax import lax
from jax.experimental import pallas as pl
from jax.experimental.pallas import tpu as pltpu
from jax.experimental.pallas import tpu_sc as plsc

B, D, H, E, G, K = 2048, 1024, 1024, 8, 64, 2
H2 = H // 2

TB = 256
NB = B // TB
T = 256
NT = (B * K) // T + E
S_PAD = NT * T
NW = 32
BPW = B // NW
LREP = 16


def _gating_body(x_ref, wg1_ref, bg1_ref, wg2_ref, bg2_ref,
                 gates_ref, g1_ref, g2_ref, pos1_ref, pos2_ref,
                 texp_ref, tval_ref,
                 carry_ref, sel1_s, sel2_s, r1_s, r2_s):
    i = pl.program_id(0)

    @pl.when(i == 0)
    def _():
        carry_ref[...] = jnp.zeros((1, E), jnp.float32)

    @pl.when(i < NB)
    def _():
        x = x_ref[...]
        h = jnp.maximum(
            jnp.dot(x, wg1_ref[...], preferred_element_type=jnp.float32)
            + bg1_ref[...], 0.0)
        logits = (jnp.dot(h, wg2_ref[...], preferred_element_type=jnp.float32)
                  + bg2_ref[...])
        m = jnp.max(logits, axis=-1, keepdims=True)
        p = jnp.exp(logits - m)
        gates = p / jnp.sum(p, axis=-1, keepdims=True)

        eidx = lax.broadcasted_iota(jnp.int32, (TB, E), 1)
        m1 = jnp.max(gates, axis=-1, keepdims=True)
        i1 = jnp.min(jnp.where(gates == m1, eidx, E), axis=-1, keepdims=True)
        sel1 = eidx == i1
        g_wo1 = jnp.where(sel1, -1.0, gates)
        m2 = jnp.max(g_wo1, axis=-1, keepdims=True)
        i2 = jnp.min(jnp.where(g_wo1 == m2, eidx, E), axis=-1, keepdims=True)
        sel2 = eidx == i2
        keep = sel1 | sel2
        gk = jnp.where(keep, gates, 0.0)
        gates_ref[...] = gk / (jnp.sum(gk, axis=-1, keepdims=True) + 1e-10)

        denom = m1 + m2 + 1e-10
        g1_ref[...] = (m1 / denom).reshape(TB)
        g2_ref[...] = (m2 / denom).reshape(TB)
        sel1_s[pl.ds(i * TB, TB), :] = sel1.astype(jnp.float32)
        sel2_s[pl.ds(i * TB, TB), :] = sel2.astype(jnp.float32)

        maskf = keep.astype(jnp.float32)
        ri = lax.broadcasted_iota(jnp.int32, (TB, TB), 0)
        ci = lax.broadcasted_iota(jnp.int32, (TB, TB), 1)
        ltri = (ri > ci).astype(jnp.bfloat16)
        rank_in = jnp.dot(ltri, maskf.astype(jnp.bfloat16),
                          preferred_element_type=jnp.float32)
        rank = rank_in + carry_ref[...]
        r1_s[pl.ds(i * TB, TB)] = jnp.sum(jnp.where(sel1, rank, 0.0), axis=-1)
        r2_s[pl.ds(i * TB, TB)] = jnp.sum(jnp.where(sel2, rank, 0.0), axis=-1)
        carry_ref[...] = carry_ref[...] + jnp.sum(maskf, axis=0, keepdims=True)

    @pl.when(i == NB)
    def _():
        cnt = carry_ref[...]
        ntiles = jnp.floor((cnt + (T - 1)) * (1.0 / T))
        ei = lax.broadcasted_iota(jnp.int32, (E, E), 0)
        ej = lax.broadcasted_iota(jnp.int32, (E, E), 1)
        u_lt = (ei < ej).astype(jnp.float32)
        u_le = (ei <= ej).astype(jnp.float32)
        cum_ex = jnp.dot(ntiles, u_lt, preferred_element_type=jnp.float32)
        cum_in = jnp.dot(ntiles, u_le, preferred_element_type=jnp.float32)
        slotbase = cum_ex * T

        trows = lax.broadcasted_iota(jnp.int32, (NT + 8, E), 0)
        acc = jnp.sum((trows >= cum_in.astype(jnp.int32)).astype(jnp.int32),
                      axis=-1)
        texp_ref[...] = jnp.minimum(acc, E - 1)
        tval_ref[...] = (acc < E).astype(jnp.int32)

        p1 = jnp.sum(sel1_s[...] * slotbase, axis=-1) + r1_s[...]
        p2 = jnp.sum(sel2_s[...] * slotbase, axis=-1) + r2_s[...]
        pos1_ref[...] = p1.astype(jnp.int32)
        pos2_ref[...] = p2.astype(jnp.int32)


def _run_gating(x, wg1, bg1, wg2, bg2, *, interpret=False):
    out_shapes = (
        jax.ShapeDtypeStruct((B, E), jnp.float32),
        jax.ShapeDtypeStruct((B,), jnp.float32),
        jax.ShapeDtypeStruct((B,), jnp.float32),
        jax.ShapeDtypeStruct((B,), jnp.int32),
        jax.ShapeDtypeStruct((B,), jnp.int32),
        jax.ShapeDtypeStruct((NT + 8,), jnp.int32),
        jax.ShapeDtypeStruct((NT + 8,), jnp.int32),
    )
    blk = lambda i: (jnp.minimum(i, NB - 1), 0)
    blk1 = lambda i: (jnp.minimum(i, NB - 1),)
    return pl.pallas_call(
        _gating_body,
        grid=(NB + 1,),
        in_specs=[
            pl.BlockSpec((TB, D), blk),
            pl.BlockSpec((D, G), lambda i: (0, 0)),
            pl.BlockSpec((1, G), lambda i: (0, 0)),
            pl.BlockSpec((G, E), lambda i: (0, 0)),
            pl.BlockSpec((1, E), lambda i: (0, 0)),
        ],
        out_specs=(
            pl.BlockSpec((TB, E), blk),
            pl.BlockSpec((TB,), blk1),
            pl.BlockSpec((TB,), blk1),
            pl.BlockSpec((B,), lambda i: (0,)),
            pl.BlockSpec((B,), lambda i: (0,)),
            pl.BlockSpec((NT + 8,), lambda i: (0,)),
            pl.BlockSpec((NT + 8,), lambda i: (0,)),
        ),
        out_shape=out_shapes,
        scratch_shapes=[
            pltpu.VMEM((1, E), jnp.float32),
            pltpu.VMEM((B, E), jnp.float32),
            pltpu.VMEM((B, E), jnp.float32),
            pltpu.VMEM((B,), jnp.float32),
            pltpu.VMEM((B,), jnp.float32),
        ],
        interpret=interpret,
    )(x, wg1, bg1.reshape(1, G), wg2, bg2.reshape(1, E))


def _dispatch_body(pos1_hbm, pos2_hbm, x_hbm, xd_hbm,
                   idx1_v, idx2_v, x_v, sem1, sem2):
    wid = lax.axis_index("c") * 16 + lax.axis_index("s")
    base = wid * BPW
    pltpu.sync_copy(pos1_hbm.at[pl.ds(base, BPW)], idx1_v)
    pltpu.sync_copy(pos2_hbm.at[pl.ds(base, BPW)], idx2_v)
    pltpu.sync_copy(x_hbm.at[pl.ds(base, BPW)], x_v)
    c1 = pltpu.async_copy(x_v, xd_hbm.at[idx1_v], sem1)
    c2 = pltpu.async_copy(x_v, xd_hbm.at[idx2_v], sem2)
    c1.wait()
    c2.wait()


def _run_dispatch(pos1, pos2, x):
    mesh = plsc.VectorSubcoreMesh(core_axis_name="c", subcore_axis_name="s")
    f = pl.kernel(
        _dispatch_body,
        out_type=jax.ShapeDtypeStruct((S_PAD, D), jnp.float32),
        mesh=mesh,
        scratch_types=(
            pltpu.VMEM((BPW,), jnp.int32),
            pltpu.VMEM((BPW,), jnp.int32),
            pltpu.VMEM((BPW, D), jnp.float32),
            pltpu.SemaphoreType.DMA,
            pltpu.SemaphoreType.DMA,
        ),
    )
    return f(pos1, pos2, x)


HH = H // 2


def _expert_body(texp_ref, tval_ref, xd_ref, w1_ref, b1_ref, w2_ref, b2_ref,
                 w3t_ref, b3_ref, out_ref):
    i = pl.program_id(0)

    @pl.when(tval_ref[i] == 1)
    def _():
        e = texp_ref[i]
        x = xd_ref[...]
        h1 = jnp.maximum(
            jnp.dot(x, w1_ref[e], preferred_element_type=jnp.float32)
            + b1_ref[0], 0.0)
        h2 = jnp.maximum(
            jnp.dot(h1, w2_ref[e], preferred_element_type=jnp.float32)
            + b2_ref[0], 0.0)
        z8 = jnp.dot(h2, w3t_ref[...], preferred_element_type=jnp.float32)
        eidx = lax.broadcasted_iota(jnp.int32, (T, E), 1)
        z = jnp.sum(jnp.where(eidx == e, z8, 0.0), axis=-1) + b3_ref[e]
        out_ref[...] = 1.0 / (1.0 + jnp.exp(-z))


def _run_experts(texp, tval, xd, w1, b1, w2, b2, w3, b3, *, interpret=False):
    grid_spec = pltpu.PrefetchScalarGridSpec(
        num_scalar_prefetch=2,
        grid=(NT,),
        in_specs=[
            pl.BlockSpec((T, D),
                         lambda i, te, tv: (jnp.where(tv[i] == 1, i, 0), 0)),
            pl.BlockSpec((E, D, H), lambda i, te, tv: (0, 0, 0)),
            pl.BlockSpec((1, 1, H), lambda i, te, tv: (te[i], 0, 0)),
            pl.BlockSpec((E, H, H2), lambda i, te, tv: (0, 0, 0)),
            pl.BlockSpec((1, 1, H2), lambda i, te, tv: (te[i], 0, 0)),
            pl.BlockSpec((H2, E), lambda i, te, tv: (0, 0)),
            pl.BlockSpec(memory_space=pltpu.SMEM),
        ],
        out_specs=pl.BlockSpec((T,), lambda i, te, tv: (i,)),
    )
    return pl.pallas_call(
        _expert_body,
        grid_spec=grid_spec,
        out_shape=jax.ShapeDtypeStruct((S_PAD,), jnp.float32),
        interpret=interpret,
    )(texp, tval, xd, w1, b1.reshape(E, 1, H), w2, b2.reshape(E, 1, H2),
      w3.reshape(E, H2).T, b3.reshape(E))


def _combine_body(o_hbm, pos1_hbm, pos2_hbm, g1_hbm, g2_hbm, pred_hbm,
                  p1_v, p2_v, o1_v, o2_v, g1_v, g2_v, out_v, sem1, sem2):
    wid = lax.axis_index("c") * 16 + lax.axis_index("s")
    base = wid * BPW
    pltpu.sync_copy(pos1_hbm.at[pl.ds(base, BPW)], p1_v)
    pltpu.sync_copy(pos2_hbm.at[pl.ds(base, BPW)], p2_v)
    pltpu.sync_copy(g1_hbm.at[pl.ds(base, BPW)], g1_v)
    pltpu.sync_copy(g2_hbm.at[pl.ds(base, BPW)], g2_v)
    c1 = pltpu.async_copy(o_hbm.at[p1_v], o1_v, sem1)
    c2 = pltpu.async_copy(o_hbm.at[p2_v], o2_v, sem2)
    c1.wait()
    c2.wait()
    for g in range(BPW // 16):
        sl = pl.ds(16 * g, 16)
        out_v[sl] = g1_v[sl] * o1_v[sl] + g2_v[sl] * o2_v[sl]
    pltpu.sync_copy(out_v, pred_hbm.at[pl.ds(base, BPW)])


def _run_combine(o_slots, pos1, pos2, g1, g2):
    mesh = plsc.VectorSubcoreMesh(core_axis_name="c", subcore_axis_name="s")
    f = pl.kernel(
        _combine_body,
        out_type=jax.ShapeDtypeStruct((B,), jnp.float32),
        mesh=mesh,
        scratch_types=(
            pltpu.VMEM((BPW,), jnp.int32),
            pltpu.VMEM((BPW,), jnp.int32),
            pltpu.VMEM((BPW,), jnp.float32),
            pltpu.VMEM((BPW,), jnp.float32),
            pltpu.VMEM((BPW,), jnp.float32),
            pltpu.VMEM((BPW,), jnp.float32),
            pltpu.VMEM((BPW,), jnp.float32),
            pltpu.SemaphoreType.DMA,
            pltpu.SemaphoreType.DMA,
        ),
    )
    return f(o_slots, pos1, pos2, g1, g2)


def kernel(inputs, Wg1, bg1, Wg2, bg2, W1, b1, W2, b2, W3, b3):
    gates, g1, g2, pos1, pos2, texp, tval = _run_gating(
        inputs, Wg1, bg1, Wg2, bg2)
    xd = _run_dispatch(pos1, pos2, inputs)
    o_slots = _run_experts(texp, tval, xd, W1, b1, W2, b2, W3, b3)
    pred = _run_combine(o_slots, pos1, pos2, g1, g2)
    return pred.reshape(B, 1), gates


# speedup vs baseline: 1.0425x; 1.0425x over previous
import functools

import jax
import jax.numpy as jnp
from jax import lax
from jax.experimental import pallas as pl
from jax.experimental.pallas import tpu as pltpu
from jax.experimental.pallas import tpu_sc as plsc

B, D, H, E, G, K = 2048, 1024, 1024, 8, 64, 2
H2 = H // 2

TB = 256
NB = B // TB
T = 256
NT = (B * K) // T + E
S_PAD = NT * T
NW = 32
BPW = B // NW


def _gating_body(x_ref, wg1_ref, bg1_ref, wg2_ref, bg2_ref,
                 gates_ref, g1_ref, g2_ref, pos1_ref, pos2_ref,
                 texp_ref, tval_ref,
                 carry_ref, sel1_s, sel2_s, r1_s, r2_s):
    i = pl.program_id(0)

    @pl.when(i == 0)
    def _():
        carry_ref[...] = jnp.zeros((1, E), jnp.float32)

    @pl.when(i < NB)
    def _():
        x = x_ref[...]
        h = jnp.maximum(
            jnp.dot(x, wg1_ref[...], preferred_element_type=jnp.float32)
            + bg1_ref[...], 0.0)
        logits = (jnp.dot(h, wg2_ref[...], preferred_element_type=jnp.float32)
                  + bg2_ref[...])
        m = jnp.max(logits, axis=-1, keepdims=True)
        p = jnp.exp(logits - m)
        gates = p / jnp.sum(p, axis=-1, keepdims=True)

        eidx = lax.broadcasted_iota(jnp.int32, (TB, E), 1)
        m1 = jnp.max(gates, axis=-1, keepdims=True)
        i1 = jnp.min(jnp.where(gates == m1, eidx, E), axis=-1, keepdims=True)
        sel1 = eidx == i1
        g_wo1 = jnp.where(sel1, -1.0, gates)
        m2 = jnp.max(g_wo1, axis=-1, keepdims=True)
        i2 = jnp.min(jnp.where(g_wo1 == m2, eidx, E), axis=-1, keepdims=True)
        sel2 = eidx == i2
        keep = sel1 | sel2
        gk = jnp.where(keep, gates, 0.0)
        gates_ref[...] = gk / (jnp.sum(gk, axis=-1, keepdims=True) + 1e-10)

        denom = m1 + m2 + 1e-10
        g1_ref[...] = (m1 / denom).reshape(TB)
        g2_ref[...] = (m2 / denom).reshape(TB)
        sel1_s[pl.ds(i * TB, TB), :] = sel1.astype(jnp.float32)
        sel2_s[pl.ds(i * TB, TB), :] = sel2.astype(jnp.float32)

        maskf = keep.astype(jnp.float32)
        ri = lax.broadcasted_iota(jnp.int32, (TB, TB), 0)
        ci = lax.broadcasted_iota(jnp.int32, (TB, TB), 1)
        ltri = (ri > ci).astype(jnp.bfloat16)
        rank_in = jnp.dot(ltri, maskf.astype(jnp.bfloat16),
                          preferred_element_type=jnp.float32)
        rank = rank_in + carry_ref[...]
        r1_s[pl.ds(i * TB, TB)] = jnp.sum(jnp.where(sel1, rank, 0.0), axis=-1)
        r2_s[pl.ds(i * TB, TB)] = jnp.sum(jnp.where(sel2, rank, 0.0), axis=-1)
        carry_ref[...] = carry_ref[...] + jnp.sum(maskf, axis=0, keepdims=True)

    @pl.when(i == NB)
    def _():
        cnt = carry_ref[...]
        ntiles = jnp.floor((cnt + (T - 1)) * (1.0 / T))
        ei = lax.broadcasted_iota(jnp.int32, (E, E), 0)
        ej = lax.broadcasted_iota(jnp.int32, (E, E), 1)
        u_lt = (ei < ej).astype(jnp.float32)
        u_le = (ei <= ej).astype(jnp.float32)
        cum_ex = jnp.dot(ntiles, u_lt, preferred_element_type=jnp.float32)
        cum_in = jnp.dot(ntiles, u_le, preferred_element_type=jnp.float32)
        slotbase = cum_ex * T

        trows = lax.broadcasted_iota(jnp.int32, (NT + 8, E), 0)
        acc = jnp.sum((trows >= cum_in.astype(jnp.int32)).astype(jnp.int32),
                      axis=-1)
        texp_ref[...] = jnp.minimum(acc, E - 1)
        tval_ref[...] = (acc < E).astype(jnp.int32)

        p1 = jnp.sum(sel1_s[...] * slotbase, axis=-1) + r1_s[...]
        p2 = jnp.sum(sel2_s[...] * slotbase, axis=-1) + r2_s[...]
        pos1_ref[...] = p1.astype(jnp.int32)
        pos2_ref[...] = p2.astype(jnp.int32)


def _run_gating(x, wg1, bg1, wg2, bg2, *, interpret=False):
    out_shapes = (
        jax.ShapeDtypeStruct((B, E), jnp.float32),
        jax.ShapeDtypeStruct((B,), jnp.float32),
        jax.ShapeDtypeStruct((B,), jnp.float32),
        jax.ShapeDtypeStruct((B,), jnp.int32),
        jax.ShapeDtypeStruct((B,), jnp.int32),
        jax.ShapeDtypeStruct((NT + 8,), jnp.int32),
        jax.ShapeDtypeStruct((NT + 8,), jnp.int32),
    )
    blk = lambda i: (jnp.minimum(i, NB - 1), 0)
    blk1 = lambda i: (jnp.minimum(i, NB - 1),)
    return pl.pallas_call(
        _gating_body,
        grid=(NB + 1,),
        in_specs=[
            pl.BlockSpec((TB, D), blk),
            pl.BlockSpec((D, G), lambda i: (0, 0)),
            pl.BlockSpec((1, G), lambda i: (0, 0)),
            pl.BlockSpec((G, E), lambda i: (0, 0)),
            pl.BlockSpec((1, E), lambda i: (0, 0)),
        ],
        out_specs=(
            pl.BlockSpec((TB, E), blk),
            pl.BlockSpec((TB,), blk1),
            pl.BlockSpec((TB,), blk1),
            pl.BlockSpec((B,), lambda i: (0,)),
            pl.BlockSpec((B,), lambda i: (0,)),
            pl.BlockSpec((NT + 8,), lambda i: (0,)),
            pl.BlockSpec((NT + 8,), lambda i: (0,)),
        ),
        out_shape=out_shapes,
        scratch_shapes=[
            pltpu.VMEM((1, E), jnp.float32),
            pltpu.VMEM((B, E), jnp.float32),
            pltpu.VMEM((B, E), jnp.float32),
            pltpu.VMEM((B,), jnp.float32),
            pltpu.VMEM((B,), jnp.float32),
        ],
        interpret=interpret,
    )(x, wg1, bg1.reshape(1, G), wg2, bg2.reshape(1, E))


def _dispatch_body(pos1_hbm, pos2_hbm, x_hbm, xd_hbm,
                   idx1_v, idx2_v, x_v, sem1, sem2):
    wid = lax.axis_index("c") * 16 + lax.axis_index("s")
    base = wid * BPW
    pltpu.sync_copy(pos1_hbm.at[pl.ds(base, BPW)], idx1_v)
    pltpu.sync_copy(pos2_hbm.at[pl.ds(base, BPW)], idx2_v)
    pltpu.sync_copy(x_hbm.at[pl.ds(base, BPW)], x_v)
    c1 = pltpu.async_copy(x_v, xd_hbm.at[idx1_v], sem1)
    c2 = pltpu.async_copy(x_v, xd_hbm.at[idx2_v], sem2)
    c1.wait()
    c2.wait()


def _run_dispatch(pos1, pos2, x):
    mesh = plsc.VectorSubcoreMesh(core_axis_name="c", subcore_axis_name="s")
    f = pl.kernel(
        _dispatch_body,
        out_type=jax.ShapeDtypeStruct((S_PAD, D), jnp.float32),
        mesh=mesh,
        scratch_types=(
            pltpu.VMEM((BPW,), jnp.int32),
            pltpu.VMEM((BPW,), jnp.int32),
            pltpu.VMEM((BPW, D), jnp.float32),
            pltpu.SemaphoreType.DMA,
            pltpu.SemaphoreType.DMA,
        ),
    )
    return f(pos1, pos2, x)


def _expert_body(texp_ref, tval_ref, xd_ref, w1_ref, b1_ref, w2_ref, b2_ref,
                 w3t_ref, b3_ref, out_ref):
    i = pl.program_id(0)

    @pl.when(tval_ref[i] == 1)
    def _():
        e = texp_ref[i]
        x = xd_ref[...]
        h1 = jnp.maximum(
            jnp.dot(x, w1_ref[0], preferred_element_type=jnp.float32)
            + b1_ref[0], 0.0)
        h2 = jnp.maximum(
            jnp.dot(h1, w2_ref[0], preferred_element_type=jnp.float32)
            + b2_ref[0], 0.0)
        z8 = jnp.dot(h2, w3t_ref[...], preferred_element_type=jnp.float32)
        eidx = lax.broadcasted_iota(jnp.int32, (T, E), 1)
        z = jnp.sum(jnp.where(eidx == e, z8, 0.0), axis=-1) + b3_ref[e]
        out_ref[...] = 1.0 / (1.0 + jnp.exp(-z))


def _run_experts(texp, tval, xd, w1, b1, w2, b2, w3, b3, *, interpret=False):
    grid_spec = pltpu.PrefetchScalarGridSpec(
        num_scalar_prefetch=2,
        grid=(NT,),
        in_specs=[
            pl.BlockSpec((T, D),
                         lambda i, te, tv: (jnp.where(tv[i] == 1, i, 0), 0)),
            pl.BlockSpec((1, D, H), lambda i, te, tv: (te[i], 0, 0)),
            pl.BlockSpec((1, 1, H), lambda i, te, tv: (te[i], 0, 0)),
            pl.BlockSpec((1, H, H2), lambda i, te, tv: (te[i], 0, 0)),
            pl.BlockSpec((1, 1, H2), lambda i, te, tv: (te[i], 0, 0)),
            pl.BlockSpec((H2, E), lambda i, te, tv: (0, 0)),
            pl.BlockSpec(memory_space=pltpu.SMEM),
        ],
        out_specs=pl.BlockSpec((T,), lambda i, te, tv: (i,)),
    )
    return pl.pallas_call(
        _expert_body,
        grid_spec=grid_spec,
        out_shape=jax.ShapeDtypeStruct((S_PAD,), jnp.float32),
        interpret=interpret,
    )(texp, tval, xd, w1, b1.reshape(E, 1, H), w2, b2.reshape(E, 1, H2),
      w3.reshape(E, H2).T, b3.reshape(E))


def _combine_body(o_hbm, pos1_hbm, pos2_hbm, g1_hbm, g2_hbm, pred_hbm,
                  p1_v, p2_v, o1_v, o2_v, g1_v, g2_v, out_v, sem1, sem2):
    wid = lax.axis_index("c") * 16 + lax.axis_index("s")
    base = wid * BPW
    pltpu.sync_copy(pos1_hbm.at[pl.ds(base, BPW)], p1_v)
    pltpu.sync_copy(pos2_hbm.at[pl.ds(base, BPW)], p2_v)
    pltpu.sync_copy(g1_hbm.at[pl.ds(base, BPW)], g1_v)
    pltpu.sync_copy(g2_hbm.at[pl.ds(base, BPW)], g2_v)
    c1 = pltpu.async_copy(o_hbm.at[p1_v], o1_v, sem1)
    c2 = pltpu.async_copy(o_hbm.at[p2_v], o2_v, sem2)
    c1.wait()
    c2.wait()
    for g in range(BPW // 16):
        sl = pl.ds(16 * g, 16)
        out_v[sl] = g1_v[sl] * o1_v[sl] + g2_v[sl] * o2_v[sl]
    pltpu.sync_copy(out_v, pred_hbm.at[pl.ds(base, BPW)])


def _run_combine(o_slots, pos1, pos2, g1, g2):
    mesh = plsc.VectorSubcoreMesh(core_axis_name="c", subcore_axis_name="s")
    f = pl.kernel(
        _combine_body,
        out_type=jax.ShapeDtypeStruct((B,), jnp.float32),
        mesh=mesh,
        scratch_types=(
            pltpu.VMEM((BPW,), jnp.int32),
            pltpu.VMEM((BPW,), jnp.int32),
            pltpu.VMEM((BPW,), jnp.float32),
            pltpu.VMEM((BPW,), jnp.float32),
            pltpu.VMEM((BPW,), jnp.float32),
            pltpu.VMEM((BPW,), jnp.float32),
            pltpu.VMEM((BPW,), jnp.float32),
            pltpu.SemaphoreType.DMA,
            pltpu.SemaphoreType.DMA,
        ),
    )
    return f(o_slots, pos1, pos2, g1, g2)


def kernel(inputs, Wg1, bg1, Wg2, bg2, W1, b1, W2, b2, W3, b3):
    gates, g1, g2, pos1, pos2, texp, tval = _run_gating(
        inputs, Wg1, bg1, Wg2, bg2)
    xd = _run_dispatch(pos1, pos2, inputs)
    o_slots = _run_experts(texp, tval, xd, W1, b1, W2, b2, W3, b3)
    pred = _run_combine(o_slots, pos1, pos2, g1, g2)
    return pred.reshape(B, 1), gates


# speedup vs baseline: 1.0538x; 1.0108x over previous
import functools

import jax
import jax.numpy as jnp
from jax import lax
from jax.experimental import pallas as pl
from jax.experimental.pallas import tpu as pltpu
from jax.experimental.pallas import tpu_sc as plsc

B, D, H, E, G, K = 2048, 1024, 1024, 8, 64, 2
H2 = H // 2

TB = 512
NB = B // TB
T = 256
NT = (B * K) // T + E
S_PAD = NT * T
NW = 32
BPW = B // NW


def _gating_body(x_ref, wg1_ref, bg1_ref, wg2_ref, bg2_ref,
                 gates_ref, g1_ref, g2_ref, pos1_ref, pos2_ref,
                 texp_ref, tval_ref,
                 carry_ref, sel1_s, sel2_s, r1_s, r2_s):
    i = pl.program_id(0)

    @pl.when(i == 0)
    def _():
        carry_ref[...] = jnp.zeros((1, E), jnp.float32)

    @pl.when(i < NB)
    def _():
        x = x_ref[...]
        h = jnp.maximum(
            jnp.dot(x, wg1_ref[...], preferred_element_type=jnp.float32)
            + bg1_ref[...], 0.0)
        logits = (jnp.dot(h, wg2_ref[...], preferred_element_type=jnp.float32)
                  + bg2_ref[...])
        m = jnp.max(logits, axis=-1, keepdims=True)
        p = jnp.exp(logits - m)
        gates = p / jnp.sum(p, axis=-1, keepdims=True)

        eidx = lax.broadcasted_iota(jnp.int32, (TB, E), 1)
        m1 = jnp.max(gates, axis=-1, keepdims=True)
        i1 = jnp.min(jnp.where(gates == m1, eidx, E), axis=-1, keepdims=True)
        sel1 = eidx == i1
        g_wo1 = jnp.where(sel1, -1.0, gates)
        m2 = jnp.max(g_wo1, axis=-1, keepdims=True)
        i2 = jnp.min(jnp.where(g_wo1 == m2, eidx, E), axis=-1, keepdims=True)
        sel2 = eidx == i2
        keep = sel1 | sel2
        gk = jnp.where(keep, gates, 0.0)
        gates_ref[...] = gk / (jnp.sum(gk, axis=-1, keepdims=True) + 1e-10)

        denom = m1 + m2 + 1e-10
        g1_ref[...] = (m1 / denom).reshape(TB)
        g2_ref[...] = (m2 / denom).reshape(TB)
        sel1_s[pl.ds(i * TB, TB), :] = sel1.astype(jnp.float32)
        sel2_s[pl.ds(i * TB, TB), :] = sel2.astype(jnp.float32)

        maskf = keep.astype(jnp.float32)
        ri = lax.broadcasted_iota(jnp.int32, (TB, TB), 0)
        ci = lax.broadcasted_iota(jnp.int32, (TB, TB), 1)
        ltri = (ri > ci).astype(jnp.float32)
        rank_in = jnp.dot(ltri, maskf,
                          preferred_element_type=jnp.float32)
        rank = rank_in + carry_ref[...]
        r1_s[pl.ds(i * TB, TB)] = jnp.sum(jnp.where(sel1, rank, 0.0), axis=-1)
        r2_s[pl.ds(i * TB, TB)] = jnp.sum(jnp.where(sel2, rank, 0.0), axis=-1)
        carry_ref[...] = carry_ref[...] + jnp.sum(maskf, axis=0, keepdims=True)

    @pl.when(i == NB)
    def _():
        cnt = carry_ref[...]
        ntiles = jnp.floor((cnt + (T - 1)) * (1.0 / T))
        ei = lax.broadcasted_iota(jnp.int32, (E, E), 0)
        ej = lax.broadcasted_iota(jnp.int32, (E, E), 1)
        u_lt = (ei < ej).astype(jnp.float32)
        u_le = (ei <= ej).astype(jnp.float32)
        cum_ex = jnp.dot(ntiles, u_lt, preferred_element_type=jnp.float32)
        cum_in = jnp.dot(ntiles, u_le, preferred_element_type=jnp.float32)
        slotbase = cum_ex * T

        trows = lax.broadcasted_iota(jnp.int32, (NT + 8, E), 0)
        acc = jnp.sum((trows >= cum_in.astype(jnp.int32)).astype(jnp.int32),
                      axis=-1)
        texp_ref[...] = jnp.minimum(acc, E - 1)
        tval_ref[...] = (acc < E).astype(jnp.int32)

        p1 = jnp.sum(sel1_s[...] * slotbase, axis=-1) + r1_s[...]
        p2 = jnp.sum(sel2_s[...] * slotbase, axis=-1) + r2_s[...]
        pos1_ref[...] = p1.astype(jnp.int32)
        pos2_ref[...] = p2.astype(jnp.int32)


def _run_gating(x, wg1, bg1, wg2, bg2, *, interpret=False):
    out_shapes = (
        jax.ShapeDtypeStruct((B, E), jnp.float32),
        jax.ShapeDtypeStruct((B,), jnp.float32),
        jax.ShapeDtypeStruct((B,), jnp.float32),
        jax.ShapeDtypeStruct((B,), jnp.int32),
        jax.ShapeDtypeStruct((B,), jnp.int32),
        jax.ShapeDtypeStruct((NT + 8,), jnp.int32),
        jax.ShapeDtypeStruct((NT + 8,), jnp.int32),
    )
    blk = lambda i: (jnp.minimum(i, NB - 1), 0)
    blk1 = lambda i: (jnp.minimum(i, NB - 1),)
    return pl.pallas_call(
        _gating_body,
        grid=(NB + 1,),
        in_specs=[
            pl.BlockSpec((TB, D), blk),
            pl.BlockSpec((D, G), lambda i: (0, 0)),
            pl.BlockSpec((1, G), lambda i: (0, 0)),
            pl.BlockSpec((G, E), lambda i: (0, 0)),
            pl.BlockSpec((1, E), lambda i: (0, 0)),
        ],
        out_specs=(
            pl.BlockSpec((TB, E), blk),
            pl.BlockSpec((TB,), blk1),
            pl.BlockSpec((TB,), blk1),
            pl.BlockSpec((B,), lambda i: (0,)),
            pl.BlockSpec((B,), lambda i: (0,)),
            pl.BlockSpec((NT + 8,), lambda i: (0,)),
            pl.BlockSpec((NT + 8,), lambda i: (0,)),
        ),
        out_shape=out_shapes,
        scratch_shapes=[
            pltpu.VMEM((1, E), jnp.float32),
            pltpu.VMEM((B, E), jnp.float32),
            pltpu.VMEM((B, E), jnp.float32),
            pltpu.VMEM((B,), jnp.float32),
            pltpu.VMEM((B,), jnp.float32),
        ],
        interpret=interpret,
    )(x, wg1, bg1.reshape(1, G), wg2, bg2.reshape(1, E))


def _dispatch_body(pos1_hbm, pos2_hbm, x_hbm, xd_hbm,
                   idx1_v, idx2_v, x_v, sem1, sem2):
    wid = lax.axis_index("c") * 16 + lax.axis_index("s")
    base = wid * BPW
    pltpu.sync_copy(pos1_hbm.at[pl.ds(base, BPW)], idx1_v)
    pltpu.sync_copy(pos2_hbm.at[pl.ds(base, BPW)], idx2_v)
    pltpu.sync_copy(x_hbm.at[pl.ds(base, BPW)], x_v)
    c1 = pltpu.async_copy(x_v, xd_hbm.at[idx1_v], sem1)
    c2 = pltpu.async_copy(x_v, xd_hbm.at[idx2_v], sem2)
    c1.wait()
    c2.wait()


def _run_dispatch(pos1, pos2, x):
    mesh = plsc.VectorSubcoreMesh(core_axis_name="c", subcore_axis_name="s")
    f = pl.kernel(
        _dispatch_body,
        out_type=jax.ShapeDtypeStruct((S_PAD, D), jnp.float32),
        mesh=mesh,
        scratch_types=(
            pltpu.VMEM((BPW,), jnp.int32),
            pltpu.VMEM((BPW,), jnp.int32),
            pltpu.VMEM((BPW, D), jnp.float32),
            pltpu.SemaphoreType.DMA,
            pltpu.SemaphoreType.DMA,
        ),
    )
    return f(pos1, pos2, x)


def _expert_body(texp_ref, tval_ref, xd_ref, w1_ref, b1_ref, w2_ref, b2_ref,
                 w3t_ref, b3_ref, out_ref):
    i = pl.program_id(0)

    @pl.when(tval_ref[i] == 1)
    def _():
        e = texp_ref[i]
        x = xd_ref[...]
        h1 = jnp.maximum(
            jnp.dot(x, w1_ref[0], preferred_element_type=jnp.float32)
            + b1_ref[0], 0.0)
        h2 = jnp.maximum(
            jnp.dot(h1, w2_ref[0], preferred_element_type=jnp.float32)
            + b2_ref[0], 0.0)
        z8 = jnp.dot(h2, w3t_ref[...], preferred_element_type=jnp.float32)
        eidx = lax.broadcasted_iota(jnp.int32, (T, E), 1)
        z = jnp.sum(jnp.where(eidx == e, z8, 0.0), axis=-1) + b3_ref[e]
        out_ref[...] = 1.0 / (1.0 + jnp.exp(-z))


def _run_experts(texp, tval, xd, w1, b1, w2, b2, w3, b3, *, interpret=False):
    grid_spec = pltpu.PrefetchScalarGridSpec(
        num_scalar_prefetch=2,
        grid=(NT,),
        in_specs=[
            pl.BlockSpec((T, D),
                         lambda i, te, tv: (jnp.where(tv[i] == 1, i, 0), 0)),
            pl.BlockSpec((1, D, H), lambda i, te, tv: (te[i], 0, 0)),
            pl.BlockSpec((1, 1, H), lambda i, te, tv: (te[i], 0, 0)),
            pl.BlockSpec((1, H, H2), lambda i, te, tv: (te[i], 0, 0)),
            pl.BlockSpec((1, 1, H2), lambda i, te, tv: (te[i], 0, 0)),
            pl.BlockSpec((H2, E), lambda i, te, tv: (0, 0)),
            pl.BlockSpec(memory_space=pltpu.SMEM),
        ],
        out_specs=pl.BlockSpec((T,), lambda i, te, tv: (i,)),
    )
    return pl.pallas_call(
        _expert_body,
        grid_spec=grid_spec,
        out_shape=jax.ShapeDtypeStruct((S_PAD,), jnp.float32),
        interpret=interpret,
    )(texp, tval, xd, w1, b1.reshape(E, 1, H), w2, b2.reshape(E, 1, H2),
      w3.reshape(E, H2).T, b3.reshape(E))


def _combine_body(o_hbm, pos1_hbm, pos2_hbm, g1_hbm, g2_hbm, pred_hbm,
                  p1_v, p2_v, o1_v, o2_v, g1_v, g2_v, out_v, sem1, sem2):
    wid = lax.axis_index("c") * 16 + lax.axis_index("s")
    base = wid * BPW
    pltpu.sync_copy(pos1_hbm.at[pl.ds(base, BPW)], p1_v)
    pltpu.sync_copy(pos2_hbm.at[pl.ds(base, BPW)], p2_v)
    pltpu.sync_copy(g1_hbm.at[pl.ds(base, BPW)], g1_v)
    pltpu.sync_copy(g2_hbm.at[pl.ds(base, BPW)], g2_v)
    c1 = pltpu.async_copy(o_hbm.at[p1_v], o1_v, sem1)
    c2 = pltpu.async_copy(o_hbm.at[p2_v], o2_v, sem2)
    c1.wait()
    c2.wait()
    for g in range(BPW // 16):
        sl = pl.ds(16 * g, 16)
        out_v[sl] = g1_v[sl] * o1_v[sl] + g2_v[sl] * o2_v[sl]
    pltpu.sync_copy(out_v, pred_hbm.at[pl.ds(base, BPW)])


def _run_combine(o_slots, pos1, pos2, g1, g2):
    mesh = plsc.VectorSubcoreMesh(core_axis_name="c", subcore_axis_name="s")
    f = pl.kernel(
        _combine_body,
        out_type=jax.ShapeDtypeStruct((B,), jnp.float32),
        mesh=mesh,
        scratch_types=(
            pltpu.VMEM((BPW,), jnp.int32),
            pltpu.VMEM((BPW,), jnp.int32),
            pltpu.VMEM((BPW,), jnp.float32),
            pltpu.VMEM((BPW,), jnp.float32),
            pltpu.VMEM((BPW,), jnp.float32),
            pltpu.VMEM((BPW,), jnp.float32),
            pltpu.VMEM((BPW,), jnp.float32),
            pltpu.SemaphoreType.DMA,
            pltpu.SemaphoreType.DMA,
        ),
    )
    return f(o_slots, pos1, pos2, g1, g2)


def kernel(inputs, Wg1, bg1, Wg2, bg2, W1, b1, W2, b2, W3, b3):
    gates, g1, g2, pos1, pos2, texp, tval = _run_gating(
        inputs, Wg1, bg1, Wg2, bg2)
    xd = _run_dispatch(pos1, pos2, inputs)
    o_slots = _run_experts(texp, tval, xd, W1, b1, W2, b2, W3, b3)
    pred = _run_combine(o_slots, pos1, pos2, g1, g2)
    return pred.reshape(B, 1), gates
